# baseline (speedup 1.0000x reference)
import dataclasses
import functools

import jax
import jax.numpy as jnp
from jax import lax
from jax.experimental import pallas as pl
from jax.experimental.pallas import tpu as pltpu
from jax.experimental.pallas import tpu_sc as plsc

N = 4096
C = 768
H = 12
HD = 64
K = 27
KP = 32
PAD = 4096
SCALE = HD ** -0.5
NW = 32
QW = N // NW
ROW_BLOCK = 512
NEG = -1e30
OB = 8


def _qkv_body(x_ref, w_ref, q_ref, kv_ref):
    y = jnp.dot(x_ref[...], w_ref[...], preferred_element_type=jnp.float32)
    q_ref[...] = y[:, :C] * SCALE
    kv_ref[...] = y[:, C:]


def _proj_body(a_ref, w_ref, b_ref, o_ref):
    o_ref[...] = (
        jnp.dot(a_ref[...], w_ref[...], preferred_element_type=jnp.float32)
        + b_ref[...]
    )


def _sc_attn_body(q_hbm, kv_hbm, cols_hbm, bias_hbm, out_hbm,
                  colsv, biasv, kvgA, kvgB, qvA, qvB, sv, av, ovA, ovB,
                  semkA, semkB, semqA, semqB, semoA, semoB):
    wid = lax.axis_index("s") * 2 + lax.axis_index("c")
    base = wid * QW
    pltpu.sync_copy(cols_hbm.at[pl.ds(base, QW)], colsv)
    pltpu.sync_copy(bias_hbm.at[pl.ds(base * KP, QW * KP)], biasv)

    lane = lax.iota(jnp.int32, 16)
    lane0 = lane == 0
    zeros = jnp.zeros((16,), jnp.float32)
    sv[pl.ds(0, 16)] = zeros
    sv[pl.ds(16, 16)] = zeros

    def compute(qi, kvg, qv, ov):
        for h in range(H):
            hd = h * HD
            q0 = qv[pl.ds(hd, 16)]
            q1 = qv[pl.ds(hd + 16, 16)]
            q2 = qv[pl.ds(hd + 32, 16)]
            q3 = qv[pl.ds(hd + 48, 16)]

            @pl.loop(0, K, unroll=3)
            def _(j):
                prod = (q0 * kvg[j, pl.ds(hd, 16)]
                        + q1 * kvg[j, pl.ds(hd + 16, 16)]
                        + q2 * kvg[j, pl.ds(hd + 32, 16)]
                        + q3 * kvg[j, pl.ds(hd + 48, 16)])
                s = jnp.sum(prod)
                plsc.store_scatter(sv, [lax.broadcast(j, (16,))],
                                   lax.broadcast(s, (16,)), mask=lane0)

            s0 = sv[pl.ds(0, 16)] + biasv[pl.ds(qi * KP, 16)]
            s1 = sv[pl.ds(16, 16)] + biasv[pl.ds(qi * KP + 16, 16)]
            m = jnp.max(jnp.maximum(s0, s1))
            e0 = jnp.exp(s0 - m)
            e1 = jnp.exp(s1 - m)
            tot = jnp.sum(e0 + e1)
            av[pl.ds(0, 16)] = e0 / tot
            av[pl.ds(16, 16)] = e1 / tot

            vd = C + hd

            @pl.loop(0, K, init_carry=(zeros, zeros, zeros, zeros),
                     unroll=3)
            def oacc(j, carry):
                o0, o1, o2, o3 = carry
                attj = plsc.load_gather(av, [lax.broadcast(j, (16,))])
                o0 = o0 + attj * kvg[j, pl.ds(vd, 16)]
                o1 = o1 + attj * kvg[j, pl.ds(vd + 16, 16)]
                o2 = o2 + attj * kvg[j, pl.ds(vd + 32, 16)]
                o3 = o3 + attj * kvg[j, pl.ds(vd + 48, 16)]
                return (o0, o1, o2, o3)

            o0, o1, o2, o3 = oacc
            ov[pl.ds(hd, 16)] = o0
            ov[pl.ds(hd + 16, 16)] = o1
            ov[pl.ds(hd + 32, 16)] = o2
            ov[pl.ds(hd + 48, 16)] = o3

    def prefetch(qi, kvg, qv, semk, semq_):
        hq = pltpu.async_copy(q_hbm.at[base + qi], qv, semq_)
        hk = pltpu.async_copy(kv_hbm.at[colsv.at[qi]], kvg, semk)
        return hq, hk

    pltpu.sync_copy(q_hbm.at[base], qvA)
    pltpu.sync_copy(kv_hbm.at[colsv.at[0]], kvgA)

    @pl.loop(0, QW, step=2)
    def _(qi):
        hqB, hkB = prefetch(qi + 1, kvgB, qvB, semkB, semqB)
        compute(qi, kvgA, qvA, ovA)
        hoA = pltpu.async_copy(ovA, out_hbm.at[base + qi], semoA)
        hqB.wait()
        hkB.wait()

        nxt = jnp.minimum(qi + 2, QW - 1)
        hqA, hkA = prefetch(nxt, kvgA, qvA, semkA, semqA)
        compute(qi + 1, kvgB, qvB, ovB)
        hoB = pltpu.async_copy(ovB, out_hbm.at[base + qi + 1], semoB)
        hoA.wait()
        hqA.wait()
        hkA.wait()
        hoB.wait()


def kernel(x, padded_neigh_inds, W_qkv, W_proj, b_proj):
    x2d = x.reshape(N, C)
    pni = padded_neigh_inds.astype(jnp.int32)
    pads = jnp.full((N, KP - K), PAD, jnp.int32)
    inds = jnp.concatenate([pni, pads], axis=1)
    cols = jnp.where(inds == PAD, 0, inds)
    bias = jnp.where(inds == PAD, NEG, 0.0).astype(jnp.float32).reshape(N * KP)

    q2d, kv2d = pl.pallas_call(
        _qkv_body,
        grid=(N // ROW_BLOCK,),
        in_specs=[
            pl.BlockSpec((ROW_BLOCK, C), lambda i: (i, 0)),
            pl.BlockSpec((C, 3 * C), lambda i: (0, 0)),
        ],
        out_specs=[
            pl.BlockSpec((ROW_BLOCK, C), lambda i: (i, 0)),
            pl.BlockSpec((ROW_BLOCK, 2 * C), lambda i: (i, 0)),
        ],
        out_shape=[
            jax.ShapeDtypeStruct((N, C), jnp.float32),
            jax.ShapeDtypeStruct((N, 2 * C), jnp.float32),
        ],
    )(x2d, W_qkv)

    mesh = plsc.VectorSubcoreMesh(
        core_axis_name="c", subcore_axis_name="s", num_cores=2, num_subcores=16
    )
    sc_params = pltpu.CompilerParams()
    if "needs_layout_passes" in pltpu.CompilerParams.__dataclass_fields__:
        sc_params = dataclasses.replace(sc_params, needs_layout_passes=False)
    attn = pl.kernel(
        _sc_attn_body,
        out_type=jax.ShapeDtypeStruct((N, C), jnp.float32),
        mesh=mesh,
        compiler_params=sc_params,
        scratch_types=[
            pltpu.VMEM((QW, KP), jnp.int32),
            pltpu.VMEM((QW * KP,), jnp.float32),
            pltpu.VMEM((KP, 2 * C), jnp.float32),
            pltpu.VMEM((KP, 2 * C), jnp.float32),
            pltpu.VMEM((C,), jnp.float32),
            pltpu.VMEM((C,), jnp.float32),
            pltpu.VMEM((KP,), jnp.float32),
            pltpu.VMEM((KP,), jnp.float32),
            pltpu.VMEM((C,), jnp.float32),
            pltpu.VMEM((C,), jnp.float32),
            pltpu.SemaphoreType.DMA,
            pltpu.SemaphoreType.DMA,
            pltpu.SemaphoreType.DMA,
            pltpu.SemaphoreType.DMA,
            pltpu.SemaphoreType.DMA,
            pltpu.SemaphoreType.DMA,
        ],
    )(q2d, kv2d, cols, bias)

    out = pl.pallas_call(
        _proj_body,
        grid=(N // ROW_BLOCK,),
        in_specs=[
            pl.BlockSpec((ROW_BLOCK, C), lambda i: (i, 0)),
            pl.BlockSpec((C, C), lambda i: (0, 0)),
            pl.BlockSpec((1, C), lambda i: (0, 0)),
        ],
        out_specs=pl.BlockSpec((ROW_BLOCK, C), lambda i: (i, 0)),
        out_shape=jax.ShapeDtypeStruct((N, C), jnp.float32),
    )(attn, W_proj, b_proj.reshape(1, C))

    return out.reshape(1, N, C)


# speedup vs baseline: 1.0719x; 1.0719x over previous
import dataclasses
import functools

import jax
import jax.numpy as jnp
from jax import lax
from jax.experimental import pallas as pl
from jax.experimental.pallas import tpu as pltpu
from jax.experimental.pallas import tpu_sc as plsc

N = 4096
C = 768
H = 12
HD = 64
K = 27
KP = 32
PAD = 4096
SCALE = HD ** -0.5
NW = 32
QW = N // NW
ROW_BLOCK = 512
NEG = -1e30
OB = 8


def _qkv_body(x_ref, wq_ref, wkv_ref, q_ref, kv_ref):
    xb = x_ref[...]
    q_ref[...] = jnp.dot(xb, wq_ref[...],
                         preferred_element_type=jnp.float32) * SCALE
    kv_ref[...] = jnp.dot(xb, wkv_ref[...],
                          preferred_element_type=jnp.float32
                          ).astype(jnp.bfloat16)


def _proj_body(a_ref, w_ref, b_ref, o_ref):
    o_ref[...] = (
        jnp.dot(a_ref[...], w_ref[...], preferred_element_type=jnp.float32)
        + b_ref[...]
    )


def _sc_attn_body(q_hbm, kv_hbm, cols_hbm, bias_hbm, out_hbm,
                  colsv, biasv, kvgA, kvgB, qvA, qvB, sv, av, ovA, ovB,
                  semkA, semkB, semqA, semqB, semoA, semoB):
    wid = lax.axis_index("s") * 2 + lax.axis_index("c")
    base = wid * QW
    pltpu.sync_copy(cols_hbm.at[pl.ds(base, QW)], colsv)
    pltpu.sync_copy(bias_hbm.at[pl.ds(base * KP, QW * KP)], biasv)

    lane = lax.iota(jnp.int32, 16)
    lane0 = lane == 0
    zeros = jnp.zeros((16,), jnp.float32)
    sv[pl.ds(0, 16)] = zeros
    sv[pl.ds(16, 16)] = zeros

    def compute(qi, kvg, qv, ov):
        for h in range(H):
            hd = h * HD
            q0 = qv[pl.ds(hd, 16)]
            q1 = qv[pl.ds(hd + 16, 16)]
            q2 = qv[pl.ds(hd + 32, 16)]
            q3 = qv[pl.ds(hd + 48, 16)]

            @pl.loop(0, K, unroll=3)
            def _(j):
                hw = h * 32
                k0, k1 = plsc.unpack(
                    plsc.bitcast(kvg[j, pl.ds(hw, 16)], jnp.bfloat16),
                    format=plsc.PackFormat.INTERLEAVED)
                k2, k3 = plsc.unpack(
                    plsc.bitcast(kvg[j, pl.ds(hw + 16, 16)], jnp.bfloat16),
                    format=plsc.PackFormat.INTERLEAVED)
                prod = q0 * k0 + q1 * k1 + q2 * k2 + q3 * k3
                s = jnp.sum(prod)
                plsc.store_scatter(sv, [lax.broadcast(j, (16,))],
                                   lax.broadcast(s, (16,)), mask=lane0)

            s0 = sv[pl.ds(0, 16)] + biasv[pl.ds(qi * KP, 16)]
            s1 = sv[pl.ds(16, 16)] + biasv[pl.ds(qi * KP + 16, 16)]
            m = jnp.max(jnp.maximum(s0, s1))
            e0 = jnp.exp(s0 - m)
            e1 = jnp.exp(s1 - m)
            tot = jnp.sum(e0 + e1)
            av[pl.ds(0, 16)] = e0 / tot
            av[pl.ds(16, 16)] = e1 / tot

            vd = C // 2 + h * 32

            @pl.loop(0, K, init_carry=(zeros, zeros, zeros, zeros),
                     unroll=3)
            def oacc(j, carry):
                o0, o1, o2, o3 = carry
                attj = plsc.load_gather(av, [lax.broadcast(j, (16,))])
                v0, v1 = plsc.unpack(
                    plsc.bitcast(kvg[j, pl.ds(vd, 16)], jnp.bfloat16),
                    format=plsc.PackFormat.INTERLEAVED)
                v2, v3 = plsc.unpack(
                    plsc.bitcast(kvg[j, pl.ds(vd + 16, 16)], jnp.bfloat16),
                    format=plsc.PackFormat.INTERLEAVED)
                o0 = o0 + attj * v0
                o1 = o1 + attj * v1
                o2 = o2 + attj * v2
                o3 = o3 + attj * v3
                return (o0, o1, o2, o3)

            o0, o1, o2, o3 = oacc
            ov[pl.ds(hd, 16)] = o0
            ov[pl.ds(hd + 16, 16)] = o1
            ov[pl.ds(hd + 32, 16)] = o2
            ov[pl.ds(hd + 48, 16)] = o3

    def prefetch(qi, kvg, qv, semk, semq_):
        hq = pltpu.async_copy(q_hbm.at[base + qi], qv, semq_)
        hk = pltpu.async_copy(kv_hbm.at[colsv.at[qi]], kvg, semk)
        return hq, hk

    pltpu.sync_copy(q_hbm.at[base], qvA)
    pltpu.sync_copy(kv_hbm.at[colsv.at[0]], kvgA)

    @pl.loop(0, QW, step=2)
    def _(qi):
        hqB, hkB = prefetch(qi + 1, kvgB, qvB, semkB, semqB)
        compute(qi, kvgA, qvA, ovA)
        hoA = pltpu.async_copy(ovA, out_hbm.at[base + qi], semoA)
        hqB.wait()
        hkB.wait()

        nxt = jnp.minimum(qi + 2, QW - 1)
        hqA, hkA = prefetch(nxt, kvgA, qvA, semkA, semqA)
        compute(qi + 1, kvgB, qvB, ovB)
        hoB = pltpu.async_copy(ovB, out_hbm.at[base + qi + 1], semoB)
        hoA.wait()
        hqA.wait()
        hkA.wait()
        hoB.wait()


def kernel(x, padded_neigh_inds, W_qkv, W_proj, b_proj):
    x2d = x.reshape(N, C)
    pni = padded_neigh_inds.astype(jnp.int32)
    pads = jnp.full((N, KP - K), PAD, jnp.int32)
    inds = jnp.concatenate([pni, pads], axis=1)
    cols = jnp.where(inds == PAD, 0, inds)
    bias = jnp.where(inds == PAD, NEG, 0.0).astype(jnp.float32).reshape(N * KP)

    Wq = W_qkv[:, :C]
    Wkv = (W_qkv[:, C:].reshape(C, 2 * C // 32, 2, 16)
           .transpose(0, 1, 3, 2).reshape(C, 2 * C))
    q2d, kv2d = pl.pallas_call(
        _qkv_body,
        grid=(N // ROW_BLOCK,),
        in_specs=[
            pl.BlockSpec((ROW_BLOCK, C), lambda i: (i, 0)),
            pl.BlockSpec((C, C), lambda i: (0, 0)),
            pl.BlockSpec((C, 2 * C), lambda i: (0, 0)),
        ],
        out_specs=[
            pl.BlockSpec((ROW_BLOCK, C), lambda i: (i, 0)),
            pl.BlockSpec((ROW_BLOCK, 2 * C), lambda i: (i, 0)),
        ],
        out_shape=[
            jax.ShapeDtypeStruct((N, C), jnp.float32),
            jax.ShapeDtypeStruct((N, 2 * C), jnp.bfloat16),
        ],
    )(x2d, Wq, Wkv)

    kv_i32 = lax.bitcast_convert_type(
        kv2d.reshape(N, C, 2), jnp.int32)
    mesh = plsc.VectorSubcoreMesh(
        core_axis_name="c", subcore_axis_name="s", num_cores=2, num_subcores=16
    )
    sc_params = pltpu.CompilerParams()
    if "needs_layout_passes" in pltpu.CompilerParams.__dataclass_fields__:
        sc_params = dataclasses.replace(sc_params, needs_layout_passes=False)
    attn = pl.kernel(
        _sc_attn_body,
        out_type=jax.ShapeDtypeStruct((N, C), jnp.float32),
        mesh=mesh,
        compiler_params=sc_params,
        scratch_types=[
            pltpu.VMEM((QW, KP), jnp.int32),
            pltpu.VMEM((QW * KP,), jnp.float32),
            pltpu.VMEM((KP, C), jnp.int32),
            pltpu.VMEM((KP, C), jnp.int32),
            pltpu.VMEM((C,), jnp.float32),
            pltpu.VMEM((C,), jnp.float32),
            pltpu.VMEM((KP,), jnp.float32),
            pltpu.VMEM((KP,), jnp.float32),
            pltpu.VMEM((C,), jnp.float32),
            pltpu.VMEM((C,), jnp.float32),
            pltpu.SemaphoreType.DMA,
            pltpu.SemaphoreType.DMA,
            pltpu.SemaphoreType.DMA,
            pltpu.SemaphoreType.DMA,
            pltpu.SemaphoreType.DMA,
            pltpu.SemaphoreType.DMA,
        ],
    )(q2d, kv_i32, cols, bias)

    out = pl.pallas_call(
        _proj_body,
        grid=(N // ROW_BLOCK,),
        in_specs=[
            pl.BlockSpec((ROW_BLOCK, C), lambda i: (i, 0)),
            pl.BlockSpec((C, C), lambda i: (0, 0)),
            pl.BlockSpec((1, C), lambda i: (0, 0)),
        ],
        out_specs=pl.BlockSpec((ROW_BLOCK, C), lambda i: (i, 0)),
        out_shape=jax.ShapeDtypeStruct((N, C), jnp.float32),
    )(attn, W_proj, b_proj.reshape(1, C))

    return out.reshape(1, N, C)
